# Initial kernel scaffold; baseline (speedup 1.0000x reference)
import jax
import jax.numpy as jnp
from jax.experimental import pallas as pl


def kernel(x, edge_index, batch, Wl1, Wr1, b1, Wl2, Wr2, b2, Wm1, bm1, Wm2, bm2):
    raise NotImplementedError("write your pallas kernel here")


# speedup vs baseline: 4.1828x; 4.1828x over previous
import functools

import jax
import jax.numpy as jnp
from jax import lax
from jax.experimental import pallas as pl
from jax.experimental.pallas import tpu as pltpu
from jax.experimental.pallas import tpu_sc as plsc

N = 10000
E = 320000
D = 128
H = 256
G = 16
C = 10

NC = 2
NS = 16
NW = NC * NS
NP = 10240
STRIPE = NP // NS
K = 80

_mesh = plsc.VectorSubcoreMesh(core_axis_name="c", subcore_axis_name="s")
_sc_params = pltpu.CompilerParams(needs_layout_passes=False)


def _zero_rows(rowsv):
    def body(i, _):
        r = i // (D // 16)
        j = i % (D // 16)
        rowsv[r, pl.ds(j * 16, 16)] = jnp.zeros((16,), jnp.float32)
        return 0
    lax.fori_loop(0, K * (D // 16), body, 0)


def _zero_acc_stripe(acc, rowsv, sid):
    def body(j, _):
        pltpu.sync_copy(rowsv, acc.at[pl.ds(sid * STRIPE + j * K, K)])
        return 0
    lax.fori_loop(0, STRIPE // K, body, 0)


def _agg1_body(x_hbm, src_hbm, dst_hbm, sum1_hbm, cnt_hbm,
               srcv, dstv, rowsv, cntv, sem, acc):
    cid = lax.axis_index("c")
    sid = lax.axis_index("s")
    wid = sid * NC + cid

    _zero_rows(rowsv)

    def zcnt(i, _):
        cntv[pl.ds(i * 16, 16)] = jnp.zeros((16,), jnp.float32)
        return 0
    lax.fori_loop(0, NP // 16, zcnt, 0)

    _zero_acc_stripe(acc, rowsv, sid)
    plsc.subcore_barrier()

    epw = E // NW
    ones16 = jnp.ones((16,), jnp.float32)

    def chunk(ci, _):
        base = wid * epw + ci * K
        pltpu.sync_copy(src_hbm.at[pl.ds(base, K)], srcv)
        pltpu.sync_copy(dst_hbm.at[pl.ds(base, K)], dstv)
        pltpu.async_copy(x_hbm.at[srcv], rowsv, sem).wait()
        pltpu.sync_copy(rowsv, acc.at[dstv], add=True)

        def hist(j, _):
            idx16 = dstv[pl.ds(j * 16, 16)]
            plsc.addupdate_scatter(cntv, [idx16], ones16)
            return 0
        lax.fori_loop(0, K // 16, hist, 0)
        return 0
    lax.fori_loop(0, epw // K, chunk, 0)

    plsc.subcore_barrier()
    pltpu.sync_copy(acc.at[pl.ds(sid * STRIPE, STRIPE)],
                    sum1_hbm.at[pl.ds(cid * NP + sid * STRIPE, STRIPE)])
    pltpu.sync_copy(cntv, cnt_hbm.at[pl.ds(wid * NP, NP)])


def _agg2_body(h2n_hbm, src_hbm, dst_hbm, sum2_hbm,
               srcv, dstv, rowsv, sem, acc):
    cid = lax.axis_index("c")
    sid = lax.axis_index("s")

    _zero_rows(rowsv)
    _zero_acc_stripe(acc, rowsv, sid)
    plsc.subcore_barrier()

    epw = E // NS
    off = cid * N

    def chunk(ci, _):
        base = sid * epw + ci * K
        pltpu.sync_copy(src_hbm.at[pl.ds(base, K)], srcv)
        pltpu.sync_copy(dst_hbm.at[pl.ds(base, K)], dstv)

        def shift(j, _):
            srcv[pl.ds(j * 16, 16)] = srcv[pl.ds(j * 16, 16)] + off
            return 0
        lax.fori_loop(0, K // 16, shift, 0)

        pltpu.async_copy(h2n_hbm.at[srcv], rowsv, sem).wait()
        pltpu.sync_copy(rowsv, acc.at[dstv], add=True)
        return 0
    lax.fori_loop(0, epw // K, chunk, 0)

    plsc.subcore_barrier()
    pltpu.sync_copy(acc.at[pl.ds(sid * STRIPE, STRIPE)],
                    sum2_hbm.at[pl.ds(cid * NP + sid * STRIPE, STRIPE)])


_agg1 = pl.kernel(
    _agg1_body,
    out_type=(jax.ShapeDtypeStruct((NC * NP, D), jnp.float32),
              jax.ShapeDtypeStruct((NW * NP,), jnp.float32)),
    mesh=_mesh,
    scratch_types=[
        pltpu.VMEM((K,), jnp.int32),
        pltpu.VMEM((K,), jnp.int32),
        pltpu.VMEM((K, D), jnp.float32),
        pltpu.VMEM((NP,), jnp.float32),
        pltpu.SemaphoreType.DMA,
        pltpu.VMEM_SHARED((NP, D), jnp.float32),
    ],
    compiler_params=_sc_params,
)

_agg2 = pl.kernel(
    _agg2_body,
    out_type=jax.ShapeDtypeStruct((NC * NP, D), jnp.float32),
    mesh=_mesh,
    scratch_types=[
        pltpu.VMEM((K,), jnp.int32),
        pltpu.VMEM((K,), jnp.int32),
        pltpu.VMEM((K, D), jnp.float32),
        pltpu.SemaphoreType.DMA,
        pltpu.VMEM_SHARED((NP, D), jnp.float32),
    ],
    compiler_params=_sc_params,
)

NB = 400
NBLK = N // NB

_HI = jax.lax.Precision.HIGHEST


def _dotT(a, b):
    return lax.dot_general(a, b, (((1,), (1,)), ((), ())), precision=_HI)


def _tc1_body(x_ref, sum1_ref, cnt_ref, wl1_ref, wr1_ref, b1_ref, out_ref):
    cnt = jnp.sum(cnt_ref[...], axis=1)
    invc = 1.0 / jnp.maximum(cnt, 1.0)
    s = sum1_ref[0] + sum1_ref[1]
    mean1 = s * invc[:, None]
    h = _dotT(mean1, wl1_ref[...]) + _dotT(x_ref[...], wr1_ref[...]) \
        + b1_ref[0][None, :]
    h = jnp.maximum(h, 0.0)
    out_ref[0] = h[:, :D]
    out_ref[1] = h[:, D:]


def _tc2_body(sum2_ref, h2n_ref, cnt_ref, batch_ref,
              wl2_ref, wr2_ref, b2_ref, wm1_ref, bm1_ref, wm2_ref, bm2_ref,
              out_ref, am2_ref, ah_ref, gcnt_ref):
    i = pl.program_id(0)

    @pl.when(i == 0)
    def _init():
        am2_ref[...] = jnp.zeros_like(am2_ref)
        ah_ref[...] = jnp.zeros_like(ah_ref)
        gcnt_ref[...] = jnp.zeros_like(gcnt_ref)

    b2d = batch_ref[...]
    gids = lax.broadcasted_iota(jnp.int32, (NB, G), 1)
    Pt = (b2d == gids).astype(jnp.float32)

    cnt = jnp.sum(cnt_ref[...], axis=1)
    invc = 1.0 / jnp.maximum(cnt, 1.0)

    def _poolT(m):
        return lax.dot_general(Pt, m, (((0,), (0,)), ((), ())), precision=_HI)

    m20 = sum2_ref[0] * invc[:, None]
    m21 = sum2_ref[1] * invc[:, None]
    am2_ref[0] += _poolT(m20)
    am2_ref[1] += _poolT(m21)
    ah_ref[0] += _poolT(h2n_ref[0])
    ah_ref[1] += _poolT(h2n_ref[1])
    gcnt_ref[0] += jnp.sum(Pt, axis=0)

    @pl.when(i == NBLK - 1)
    def _fin():
        ginv = 1.0 / jnp.maximum(gcnt_ref[0], 1.0)
        wl2 = wl2_ref[...]
        wr2 = wr2_ref[...]
        g = (_dotT(am2_ref[0] * ginv[:, None], wl2[:, :D])
             + _dotT(am2_ref[1] * ginv[:, None], wl2[:, D:])
             + _dotT(ah_ref[0] * ginv[:, None], wr2[:, :D])
             + _dotT(ah_ref[1] * ginv[:, None], wr2[:, D:])
             + b2_ref[0][None, :])
        m = jnp.maximum(_dotT(g, wm1_ref[...]) + bm1_ref[0][None, :], 0.0)
        out_ref[...] = _dotT(m, wm2_ref[...]) + bm2_ref[0][None, :]


def kernel(x, edge_index, batch, Wl1, Wr1, b1, Wl2, Wr2, b2, Wm1, bm1, Wm2, bm2):
    src = edge_index[0]
    dst = edge_index[1]

    sum1_flat, cnt_flat = _agg1(x, src, dst)
    sum1 = sum1_flat.reshape(NC, NP, D)
    cnt = cnt_flat.reshape(NW, NP).T

    h2n = pl.pallas_call(
        _tc1_body,
        grid=(NBLK,),
        in_specs=[
            pl.BlockSpec((NB, D), lambda i: (i, 0)),
            pl.BlockSpec((NC, NB, D), lambda i: (0, i, 0)),
            pl.BlockSpec((NB, NW), lambda i: (i, 0)),
            pl.BlockSpec((H, D), lambda i: (0, 0)),
            pl.BlockSpec((H, D), lambda i: (0, 0)),
            pl.BlockSpec((1, H), lambda i: (0, 0)),
        ],
        out_specs=pl.BlockSpec((NC, NB, D), lambda i: (0, i, 0)),
        out_shape=jax.ShapeDtypeStruct((NC, N, D), jnp.float32),
    )(x, sum1, cnt, Wl1, Wr1, b1.reshape(1, H))

    sum2_flat = _agg2(h2n.reshape(NC * N, D), src, dst)
    sum2 = sum2_flat.reshape(NC, NP, D)

    out = pl.pallas_call(
        _tc2_body,
        grid=(NBLK,),
        in_specs=[
            pl.BlockSpec((NC, NB, D), lambda i: (0, i, 0)),
            pl.BlockSpec((NC, NB, D), lambda i: (0, i, 0)),
            pl.BlockSpec((NB, NW), lambda i: (i, 0)),
            pl.BlockSpec((NB, 1), lambda i: (i, 0)),
            pl.BlockSpec((H, H), lambda i: (0, 0)),
            pl.BlockSpec((H, H), lambda i: (0, 0)),
            pl.BlockSpec((1, H), lambda i: (0, 0)),
            pl.BlockSpec((H, H), lambda i: (0, 0)),
            pl.BlockSpec((1, H), lambda i: (0, 0)),
            pl.BlockSpec((C, H), lambda i: (0, 0)),
            pl.BlockSpec((1, C), lambda i: (0, 0)),
        ],
        out_specs=pl.BlockSpec((G, C), lambda i: (0, 0)),
        out_shape=jax.ShapeDtypeStruct((G, C), jnp.float32),
        scratch_shapes=[
            pltpu.VMEM((NC, G, D), jnp.float32),
            pltpu.VMEM((NC, G, D), jnp.float32),
            pltpu.VMEM((1, G), jnp.float32),
        ],
    )(sum2, h2n, cnt, batch.reshape(N, 1),
      Wl2, Wr2, b2.reshape(1, H), Wm1, bm1.reshape(1, H),
      Wm2, bm2.reshape(1, C))

    return out


# speedup vs baseline: 11.1725x; 2.6711x over previous
import jax
import jax.numpy as jnp
from jax import lax
from jax.experimental import pallas as pl
from jax.experimental.pallas import tpu as pltpu
from jax.experimental.pallas import tpu_sc as plsc

N = 10000
E = 320000
D = 128
H = 256
G = 16
C = 10

NC = 2
NS = 16
NW = NC * NS
NP = 10240
STRIPE = NP // NS
K = 80
NBUF = 4
AI = 3
AG = 2
LAG = 1

_mesh = plsc.VectorSubcoreMesh(core_axis_name="c", subcore_axis_name="s")
_sc_params = pltpu.CompilerParams(needs_layout_passes=False)


def _zero_rows(rowsv):
    def body(i, _):
        r = i // (D // 16)
        j = i % (D // 16)
        rowsv[r, pl.ds(j * 16, 16)] = jnp.zeros((16,), jnp.float32)
        return 0
    lax.fori_loop(0, K * (D // 16), body, 0)


def _zero_acc_stripe(acc, rowsv, sid):
    def body(j, _):
        pltpu.sync_copy(rowsv, acc.at[pl.ds(sid * STRIPE + j * K, K)])
        return 0
    lax.fori_loop(0, STRIPE // K, body, 0)


def _agg_pass(table_hbm, src_hbm, src_base, dst_hbm, dst_base, acc,
              srcvs, dstvs, bufs, xsems, dsems, gsems, ssems, nch):
    def i_start(i, b):
        pltpu.async_copy(src_hbm.at[pl.ds(src_base + i * K, K)],
                         srcvs[b], xsems[b])
        pltpu.async_copy(dst_hbm.at[pl.ds(dst_base + i * K, K)],
                         dstvs[b], dsems[b])

    def x_wait(i, b):
        pltpu.make_async_copy(src_hbm.at[pl.ds(src_base + i * K, K)],
                              srcvs[b], xsems[b]).wait()

    def d_wait(i, b):
        pltpu.make_async_copy(dst_hbm.at[pl.ds(dst_base + i * K, K)],
                              dstvs[b], dsems[b]).wait()

    def g_start(i, b):
        pltpu.async_copy(table_hbm.at[srcvs[b]], bufs[b], gsems[b])

    def g_wait(i, b):
        pltpu.make_async_copy(table_hbm.at[srcvs[b]], bufs[b],
                              gsems[b]).wait()

    def s_start(i, b):
        pltpu.async_copy(bufs[b], acc.at[dstvs[b]], ssems[b], add=True)

    def s_wait(i, b):
        pltpu.make_async_copy(bufs[b], acc.at[dstvs[b]], ssems[b]).wait()

    for j in range(AI):
        i_start(j, j)
    for j in range(AG):
        x_wait(j, j)
        g_start(j, j)

    def substep(i, b, static=False):
        g_wait(i, b)
        d_wait(i, b)
        s_start(i, b)
        bi = (b + AI) % NBUF
        bg = (b + AG) % NBUF

        def waits():
            s_wait(i - LAG, (b + NBUF - LAG) % NBUF)
        if static:
            if i >= LAG:
                waits()
            if i + AI < nch:
                i_start(i + AI, bi)
            if i + AG < nch:
                x_wait(i + AG, bg)
                g_start(i + AG, bg)
        else:
            pl.when(i >= LAG)(waits)
            pl.when(i + AI < nch)(lambda: i_start(i + AI, bi))

            def adv_g():
                x_wait(i + AG, bg)
                g_start(i + AG, bg)
            pl.when(i + AG < nch)(adv_g)

    def step(g, _):
        for b in range(NBUF):
            substep(g * NBUF + b, b)
        return 0
    lax.fori_loop(0, nch // NBUF, step, 0)

    for i in range((nch // NBUF) * NBUF, nch):
        substep(i, i % NBUF, static=True)

    for i in range(nch - LAG, nch):
        s_wait(i, i % NBUF)


_NCH1 = E // NW // K
_NCH2 = E // NS // K


def _agg1_body(x_hbm, src_hbm, dst_hbm, sum1_hbm, *scr):
    srcvs, dstvs, bufs = scr[0:NBUF], scr[NBUF:2 * NBUF], scr[2 * NBUF:3 * NBUF]
    sems = scr[3 * NBUF:7 * NBUF]
    acc = scr[7 * NBUF]
    xsems, dsems = sems[0:NBUF], sems[NBUF:2 * NBUF]
    gsems, ssems = sems[2 * NBUF:3 * NBUF], sems[3 * NBUF:4 * NBUF]

    cid = lax.axis_index("c")
    sid = lax.axis_index("s")
    wid = sid * NC + cid

    _zero_rows(bufs[0])
    _zero_acc_stripe(acc, bufs[0], sid)
    plsc.subcore_barrier()

    base = wid * _NCH1 * K
    _agg_pass(x_hbm, src_hbm, base, dst_hbm, base, acc,
              srcvs, dstvs, bufs, xsems, dsems, gsems, ssems, _NCH1)

    plsc.subcore_barrier()
    pltpu.sync_copy(acc.at[pl.ds(sid * STRIPE, STRIPE)],
                    sum1_hbm.at[pl.ds(cid * NP + sid * STRIPE, STRIPE)])


def _agg2_body(h2n_hbm, src2_hbm, dst_hbm, sum2_hbm, *scr):
    srcvs, dstvs, bufs = scr[0:NBUF], scr[NBUF:2 * NBUF], scr[2 * NBUF:3 * NBUF]
    sems = scr[3 * NBUF:7 * NBUF]
    acc = scr[7 * NBUF]
    xsems, dsems = sems[0:NBUF], sems[NBUF:2 * NBUF]
    gsems, ssems = sems[2 * NBUF:3 * NBUF], sems[3 * NBUF:4 * NBUF]

    cid = lax.axis_index("c")
    sid = lax.axis_index("s")

    _zero_rows(bufs[0])
    _zero_acc_stripe(acc, bufs[0], sid)
    plsc.subcore_barrier()

    _agg_pass(h2n_hbm, src2_hbm, cid * E + sid * _NCH2 * K,
              dst_hbm, sid * _NCH2 * K, acc,
              srcvs, dstvs, bufs, xsems, dsems, gsems, ssems, _NCH2)

    plsc.subcore_barrier()
    pltpu.sync_copy(acc.at[pl.ds(sid * STRIPE, STRIPE)],
                    sum2_hbm.at[pl.ds(cid * NP + sid * STRIPE, STRIPE)])


def _cnt_body(dst_hbm, cnt_hbm, dstv, cntv):
    cid = lax.axis_index("c")
    sid = lax.axis_index("s")
    wid = sid * NC + cid
    epw = E // NW

    def zcnt(i, _):
        cntv[pl.ds(i * 16, 16)] = jnp.zeros((16,), jnp.float32)
        return 0
    lax.fori_loop(0, NP // 16, zcnt, 0)

    pltpu.sync_copy(dst_hbm.at[pl.ds(wid * epw, epw)], dstv)
    ones16 = jnp.ones((16,), jnp.float32)

    def h16(j, _):
        idx16 = dstv[pl.ds(j * 16, 16)]
        plsc.addupdate_scatter(cntv, [idx16], ones16)
        return 0
    lax.fori_loop(0, epw // 16, h16, 0)

    pltpu.sync_copy(cntv, cnt_hbm.at[pl.ds(wid * NP, NP)])


def _sc_scratch():
    s = [pltpu.VMEM((K,), jnp.int32)] * (2 * NBUF)
    s += [pltpu.VMEM((K, D), jnp.float32)] * NBUF
    s += [pltpu.SemaphoreType.DMA] * (4 * NBUF)
    s += [pltpu.VMEM_SHARED((NP, D), jnp.float32)]
    return s


_agg1 = pl.kernel(
    _agg1_body,
    out_type=jax.ShapeDtypeStruct((NC * NP, D), jnp.float32),
    mesh=_mesh,
    scratch_types=_sc_scratch(),
    compiler_params=_sc_params,
)

_agg2 = pl.kernel(
    _agg2_body,
    out_type=jax.ShapeDtypeStruct((NC * NP, D), jnp.float32),
    mesh=_mesh,
    scratch_types=_sc_scratch(),
    compiler_params=_sc_params,
)

_cnt_kernel = pl.kernel(
    _cnt_body,
    out_type=jax.ShapeDtypeStruct((NW * NP,), jnp.float32),
    mesh=_mesh,
    scratch_types=[
        pltpu.VMEM((E // NW,), jnp.int32),
        pltpu.VMEM((NP,), jnp.float32),
    ],
    compiler_params=_sc_params,
)

NB = 400
NBLK = N // NB

_HI = jax.lax.Precision.HIGHEST


def _dotT(a, b):
    return lax.dot_general(a, b, (((1,), (1,)), ((), ())), precision=_HI)


def _tc1_body(x_ref, sum1_ref, cnt_ref, wl1_ref, wr1_ref, b1_ref, out_ref):
    cnt = jnp.sum(cnt_ref[...], axis=1)
    invc = 1.0 / jnp.maximum(cnt, 1.0)
    s = sum1_ref[0] + sum1_ref[1]
    mean1 = s * invc[:, None]
    h = _dotT(mean1, wl1_ref[...]) + _dotT(x_ref[...], wr1_ref[...]) \
        + b1_ref[0][None, :]
    h = jnp.maximum(h, 0.0)
    out_ref[0] = h[:, :D]
    out_ref[1] = h[:, D:]


def _tc2_body(sum2_ref, h2n_ref, cnt_ref, batch_ref,
              wl2_ref, wr2_ref, b2_ref, wm1_ref, bm1_ref, wm2_ref, bm2_ref,
              out_ref, am2_ref, ah_ref, gcnt_ref):
    i = pl.program_id(0)

    @pl.when(i == 0)
    def _init():
        am2_ref[...] = jnp.zeros_like(am2_ref)
        ah_ref[...] = jnp.zeros_like(ah_ref)
        gcnt_ref[...] = jnp.zeros_like(gcnt_ref)

    b2d = batch_ref[...]
    gids = lax.broadcasted_iota(jnp.int32, (NB, G), 1)
    Pt = (b2d == gids).astype(jnp.float32)

    cnt = jnp.sum(cnt_ref[...], axis=1)
    invc = 1.0 / jnp.maximum(cnt, 1.0)

    def _poolT(m):
        return lax.dot_general(Pt, m, (((0,), (0,)), ((), ())), precision=_HI)

    am2_ref[0] += _poolT(sum2_ref[0] * invc[:, None])
    am2_ref[1] += _poolT(sum2_ref[1] * invc[:, None])
    ah_ref[0] += _poolT(h2n_ref[0])
    ah_ref[1] += _poolT(h2n_ref[1])
    gcnt_ref[0] += jnp.sum(Pt, axis=0)

    @pl.when(i == NBLK - 1)
    def _fin():
        ginv = 1.0 / jnp.maximum(gcnt_ref[0], 1.0)
        wl2 = wl2_ref[...]
        wr2 = wr2_ref[...]
        g = (_dotT(am2_ref[0] * ginv[:, None], wl2[:, :D])
             + _dotT(am2_ref[1] * ginv[:, None], wl2[:, D:])
             + _dotT(ah_ref[0] * ginv[:, None], wr2[:, :D])
             + _dotT(ah_ref[1] * ginv[:, None], wr2[:, D:])
             + b2_ref[0][None, :])
        m = jnp.maximum(_dotT(g, wm1_ref[...]) + bm1_ref[0][None, :], 0.0)
        out_ref[...] = _dotT(m, wm2_ref[...]) + bm2_ref[0][None, :]


def kernel(x, edge_index, batch, Wl1, Wr1, b1, Wl2, Wr2, b2, Wm1, bm1, Wm2, bm2):
    src = edge_index[0]
    dst = edge_index[1]
    srcB = jnp.concatenate([src, src + N])

    cnt_flat = _cnt_kernel(dst)
    sum1_flat = _agg1(x, src, dst)
    sum1 = sum1_flat.reshape(NC, NP, D)
    cnt = cnt_flat.reshape(NW, NP).T

    h2n = pl.pallas_call(
        _tc1_body,
        grid=(NBLK,),
        in_specs=[
            pl.BlockSpec((NB, D), lambda i: (i, 0)),
            pl.BlockSpec((NC, NB, D), lambda i: (0, i, 0)),
            pl.BlockSpec((NB, NW), lambda i: (i, 0)),
            pl.BlockSpec((H, D), lambda i: (0, 0)),
            pl.BlockSpec((H, D), lambda i: (0, 0)),
            pl.BlockSpec((1, H), lambda i: (0, 0)),
        ],
        out_specs=pl.BlockSpec((NC, NB, D), lambda i: (0, i, 0)),
        out_shape=jax.ShapeDtypeStruct((NC, N, D), jnp.float32),
    )(x, sum1, cnt, Wl1, Wr1, b1.reshape(1, H))

    sum2_flat = _agg2(h2n.reshape(NC * N, D), srcB, dst)
    sum2 = sum2_flat.reshape(NC, NP, D)

    out = pl.pallas_call(
        _tc2_body,
        grid=(NBLK,),
        in_specs=[
            pl.BlockSpec((NC, NB, D), lambda i: (0, i, 0)),
            pl.BlockSpec((NC, NB, D), lambda i: (0, i, 0)),
            pl.BlockSpec((NB, NW), lambda i: (i, 0)),
            pl.BlockSpec((NB, 1), lambda i: (i, 0)),
            pl.BlockSpec((H, H), lambda i: (0, 0)),
            pl.BlockSpec((H, H), lambda i: (0, 0)),
            pl.BlockSpec((1, H), lambda i: (0, 0)),
            pl.BlockSpec((H, H), lambda i: (0, 0)),
            pl.BlockSpec((1, H), lambda i: (0, 0)),
            pl.BlockSpec((C, H), lambda i: (0, 0)),
            pl.BlockSpec((1, C), lambda i: (0, 0)),
        ],
        out_specs=pl.BlockSpec((G, C), lambda i: (0, 0)),
        out_shape=jax.ShapeDtypeStruct((G, C), jnp.float32),
        scratch_shapes=[
            pltpu.VMEM((NC, G, D), jnp.float32),
            pltpu.VMEM((NC, G, D), jnp.float32),
            pltpu.VMEM((1, G), jnp.float32),
        ],
    )(sum2, h2n, cnt, batch.reshape(N, 1),
      Wl2, Wr2, b2.reshape(1, H), Wm1, bm1.reshape(1, H),
      Wm2, bm2.reshape(1, C))

    return out


# speedup vs baseline: 11.8359x; 1.0594x over previous
import jax
import jax.numpy as jnp
from jax import lax
from jax.experimental import pallas as pl
from jax.experimental.pallas import tpu as pltpu
from jax.experimental.pallas import tpu_sc as plsc

N = 10000
E = 320000
D = 128
H = 256
G = 16
C = 10

NC = 2
NS = 16
NW = NC * NS
NP = 10240
STRIPE = NP // NS
K = 80
NBUF = 4
AI = 3
AG = 2
LAG = 1

_mesh = plsc.VectorSubcoreMesh(core_axis_name="c", subcore_axis_name="s")
_sc_params = pltpu.CompilerParams(needs_layout_passes=False)


def _zero_rows(rowsv):
    def body(i, _):
        r = i // (D // 16)
        j = i % (D // 16)
        rowsv[r, pl.ds(j * 16, 16)] = jnp.zeros((16,), jnp.float32)
        return 0
    lax.fori_loop(0, K * (D // 16), body, 0)


def _zero_acc_stripe(acc, rowsv, sid):
    def body(j, _):
        pltpu.sync_copy(rowsv, acc.at[pl.ds(sid * STRIPE + j * K, K)])
        return 0
    lax.fori_loop(0, STRIPE // K, body, 0)


def _agg_pass(table_hbm, src_hbm, src_base, dst_hbm, dst_base, acc,
              srcvs, dstvs, bufs, xsems, dsems, gsems, ssems, nch):
    def i_start(i, b):
        pltpu.async_copy(src_hbm.at[pl.ds(src_base + i * K, K)],
                         srcvs[b], xsems[b])
        pltpu.async_copy(dst_hbm.at[pl.ds(dst_base + i * K, K)],
                         dstvs[b], dsems[b])

    def x_wait(i, b):
        pltpu.make_async_copy(src_hbm.at[pl.ds(src_base + i * K, K)],
                              srcvs[b], xsems[b]).wait()

    def d_wait(i, b):
        pltpu.make_async_copy(dst_hbm.at[pl.ds(dst_base + i * K, K)],
                              dstvs[b], dsems[b]).wait()

    def g_start(i, b):
        pltpu.async_copy(table_hbm.at[srcvs[b]], bufs[b], gsems[b])

    def g_wait(i, b):
        pltpu.make_async_copy(table_hbm.at[srcvs[b]], bufs[b],
                              gsems[b]).wait()

    def s_start(i, b):
        pltpu.async_copy(bufs[b], acc.at[dstvs[b]], ssems[b], add=True)

    def s_wait(i, b):
        pltpu.make_async_copy(bufs[b], acc.at[dstvs[b]], ssems[b]).wait()

    for j in range(AI):
        i_start(j, j)
    for j in range(AG):
        x_wait(j, j)
        g_start(j, j)

    def substep(i, b, static=False):
        g_wait(i, b)
        d_wait(i, b)
        s_start(i, b)
        bi = (b + AI) % NBUF
        bg = (b + AG) % NBUF

        def waits():
            s_wait(i - LAG, (b + NBUF - LAG) % NBUF)
        if static:
            if i >= LAG:
                waits()
            if i + AI < nch:
                i_start(i + AI, bi)
            if i + AG < nch:
                x_wait(i + AG, bg)
                g_start(i + AG, bg)
        else:
            pl.when(i >= LAG)(waits)
            pl.when(i + AI < nch)(lambda: i_start(i + AI, bi))

            def adv_g():
                x_wait(i + AG, bg)
                g_start(i + AG, bg)
            pl.when(i + AG < nch)(adv_g)

    def step(g, _):
        for b in range(NBUF):
            substep(g * NBUF + b, b)
        return 0
    lax.fori_loop(0, nch // NBUF, step, 0)

    for i in range((nch // NBUF) * NBUF, nch):
        substep(i, i % NBUF, static=True)

    for i in range(nch - LAG, nch):
        s_wait(i, i % NBUF)


_NCH1 = E // NW // K
_NCH2 = E // NS // K


def _agg1_body(x_hbm, ei_hbm, sum1_hbm, *scr):
    srcvs, dstvs, bufs = scr[0:NBUF], scr[NBUF:2 * NBUF], scr[2 * NBUF:3 * NBUF]
    sems = scr[3 * NBUF:7 * NBUF]
    acc = scr[7 * NBUF]
    xsems, dsems = sems[0:NBUF], sems[NBUF:2 * NBUF]
    gsems, ssems = sems[2 * NBUF:3 * NBUF], sems[3 * NBUF:4 * NBUF]

    cid = lax.axis_index("c")
    sid = lax.axis_index("s")
    wid = sid * NC + cid

    _zero_rows(bufs[0])
    _zero_acc_stripe(acc, bufs[0], sid)
    plsc.subcore_barrier()

    base = wid * _NCH1 * K
    _agg_pass(x_hbm, ei_hbm, base, ei_hbm, E + base, acc,
              srcvs, dstvs, bufs, xsems, dsems, gsems, ssems, _NCH1)

    plsc.subcore_barrier()
    pltpu.sync_copy(acc.at[pl.ds(sid * STRIPE, STRIPE)],
                    sum1_hbm.at[pl.ds(cid * NP + sid * STRIPE, STRIPE)])


def _agg2_body(h0_hbm, h1_hbm, ei_hbm, sum2_hbm, *scr):
    srcvs, dstvs, bufs = scr[0:NBUF], scr[NBUF:2 * NBUF], scr[2 * NBUF:3 * NBUF]
    sems = scr[3 * NBUF:7 * NBUF]
    acc = scr[7 * NBUF]
    xsems, dsems = sems[0:NBUF], sems[NBUF:2 * NBUF]
    gsems, ssems = sems[2 * NBUF:3 * NBUF], sems[3 * NBUF:4 * NBUF]

    cid = lax.axis_index("c")
    sid = lax.axis_index("s")

    _zero_rows(bufs[0])
    _zero_acc_stripe(acc, bufs[0], sid)
    plsc.subcore_barrier()

    base = sid * _NCH2 * K

    @pl.when(cid == 0)
    def _():
        _agg_pass(h0_hbm, ei_hbm, base, ei_hbm, E + base, acc,
                  srcvs, dstvs, bufs, xsems, dsems, gsems, ssems, _NCH2)

    @pl.when(cid == 1)
    def _():
        _agg_pass(h1_hbm, ei_hbm, base, ei_hbm, E + base, acc,
                  srcvs, dstvs, bufs, xsems, dsems, gsems, ssems, _NCH2)

    plsc.subcore_barrier()
    pltpu.sync_copy(acc.at[pl.ds(sid * STRIPE, STRIPE)],
                    sum2_hbm.at[pl.ds(cid * NP + sid * STRIPE, STRIPE)])


def _cnt_body(ei_hbm, cnt_hbm, dstv, cntv):
    cid = lax.axis_index("c")
    sid = lax.axis_index("s")
    wid = sid * NC + cid
    epw = E // NW

    def zcnt(i, _):
        cntv[pl.ds(i * 16, 16)] = jnp.zeros((16,), jnp.float32)
        return 0
    lax.fori_loop(0, NP // 16, zcnt, 0)

    pltpu.sync_copy(ei_hbm.at[pl.ds(E + wid * epw, epw)], dstv)
    ones16 = jnp.ones((16,), jnp.float32)

    def h16(j, _):
        idx16 = dstv[pl.ds(j * 16, 16)]
        plsc.addupdate_scatter(cntv, [idx16], ones16)
        return 0
    lax.fori_loop(0, epw // 16, h16, 0)

    pltpu.sync_copy(cntv, cnt_hbm.at[pl.ds(wid * NP, NP)])


def _sc_scratch():
    s = [pltpu.VMEM((K,), jnp.int32)] * (2 * NBUF)
    s += [pltpu.VMEM((K, D), jnp.float32)] * NBUF
    s += [pltpu.SemaphoreType.DMA] * (4 * NBUF)
    s += [pltpu.VMEM_SHARED((NP, D), jnp.float32)]
    return s


_agg1 = pl.kernel(
    _agg1_body,
    out_type=jax.ShapeDtypeStruct((NC * NP, D), jnp.float32),
    mesh=_mesh,
    scratch_types=_sc_scratch(),
    compiler_params=_sc_params,
)

_agg2 = pl.kernel(
    _agg2_body,
    out_type=jax.ShapeDtypeStruct((NC * NP, D), jnp.float32),
    mesh=_mesh,
    scratch_types=_sc_scratch(),
    compiler_params=_sc_params,
)

_cnt_kernel = pl.kernel(
    _cnt_body,
    out_type=jax.ShapeDtypeStruct((NW * NP,), jnp.float32),
    mesh=_mesh,
    scratch_types=[
        pltpu.VMEM((E // NW,), jnp.int32),
        pltpu.VMEM((NP,), jnp.float32),
    ],
    compiler_params=_sc_params,
)

NB = 400
NBLK = N // NB

_HI = jax.lax.Precision.DEFAULT


def _dotT(a, b):
    return lax.dot_general(a, b, (((1,), (1,)), ((), ())), precision=_HI)


def _tc1_body(x_ref, sum1_ref, cnt_ref, wl1_ref, wr1_ref, b1_ref,
              out0_ref, out1_ref):
    cnt = jnp.sum(cnt_ref[...], axis=1)
    invc = 1.0 / jnp.maximum(cnt, 1.0)
    s = sum1_ref[0] + sum1_ref[1]
    mean1 = s * invc[:, None]
    h = _dotT(mean1, wl1_ref[...]) + _dotT(x_ref[...], wr1_ref[...]) \
        + b1_ref[0][None, :]
    h = jnp.maximum(h, 0.0)
    out0_ref[...] = h[:, :D]
    out1_ref[...] = h[:, D:]


def _tc2_body(sum2_ref, h0_ref, h1_ref, cnt_ref, batch_ref,
              wl2_ref, wr2_ref, b2_ref, wm1_ref, bm1_ref, wm2_ref, bm2_ref,
              out_ref, am2_ref, ah_ref, gcnt_ref):
    i = pl.program_id(0)

    @pl.when(i == 0)
    def _init():
        am2_ref[...] = jnp.zeros_like(am2_ref)
        ah_ref[...] = jnp.zeros_like(ah_ref)
        gcnt_ref[...] = jnp.zeros_like(gcnt_ref)

    b2d = batch_ref[...]
    gids = lax.broadcasted_iota(jnp.int32, (NB, G), 1)
    Pt = (b2d == gids).astype(jnp.float32)

    cnt = jnp.sum(cnt_ref[...], axis=1)
    invc = 1.0 / jnp.maximum(cnt, 1.0)

    def _poolT(m):
        return lax.dot_general(Pt, m, (((0,), (0,)), ((), ())), precision=_HI)

    am2_ref[0] += _poolT(sum2_ref[0] * invc[:, None])
    am2_ref[1] += _poolT(sum2_ref[1] * invc[:, None])
    ah_ref[0] += _poolT(h0_ref[...])
    ah_ref[1] += _poolT(h1_ref[...])
    gcnt_ref[0] += jnp.sum(Pt, axis=0)

    @pl.when(i == NBLK - 1)
    def _fin():
        ginv = 1.0 / jnp.maximum(gcnt_ref[0], 1.0)
        wl2 = wl2_ref[...]
        wr2 = wr2_ref[...]
        g = (_dotT(am2_ref[0] * ginv[:, None], wl2[:, :D])
             + _dotT(am2_ref[1] * ginv[:, None], wl2[:, D:])
             + _dotT(ah_ref[0] * ginv[:, None], wr2[:, :D])
             + _dotT(ah_ref[1] * ginv[:, None], wr2[:, D:])
             + b2_ref[0][None, :])
        m = jnp.maximum(_dotT(g, wm1_ref[...]) + bm1_ref[0][None, :], 0.0)
        out_ref[...] = _dotT(m, wm2_ref[...]) + bm2_ref[0][None, :]


def kernel(x, edge_index, batch, Wl1, Wr1, b1, Wl2, Wr2, b2, Wm1, bm1, Wm2, bm2):
    ei = edge_index.reshape(2 * E)

    cnt_flat = _cnt_kernel(ei)
    sum1_flat = _agg1(x, ei)
    sum1 = sum1_flat.reshape(NC, NP, D)
    cnt = cnt_flat.reshape(NW, NP).T

    h0, h1 = pl.pallas_call(
        _tc1_body,
        grid=(NBLK,),
        in_specs=[
            pl.BlockSpec((NB, D), lambda i: (i, 0)),
            pl.BlockSpec((NC, NB, D), lambda i: (0, i, 0)),
            pl.BlockSpec((NB, NW), lambda i: (i, 0)),
            pl.BlockSpec((H, D), lambda i: (0, 0)),
            pl.BlockSpec((H, D), lambda i: (0, 0)),
            pl.BlockSpec((1, H), lambda i: (0, 0)),
        ],
        out_specs=[pl.BlockSpec((NB, D), lambda i: (i, 0)),
                   pl.BlockSpec((NB, D), lambda i: (i, 0))],
        out_shape=[jax.ShapeDtypeStruct((N, D), jnp.float32),
                   jax.ShapeDtypeStruct((N, D), jnp.float32)],
    )(x, sum1, cnt, Wl1, Wr1, b1.reshape(1, H))

    sum2_flat = _agg2(h0, h1, ei)
    sum2 = sum2_flat.reshape(NC, NP, D)

    out = pl.pallas_call(
        _tc2_body,
        grid=(NBLK,),
        in_specs=[
            pl.BlockSpec((NC, NB, D), lambda i: (0, i, 0)),
            pl.BlockSpec((NB, D), lambda i: (i, 0)),
            pl.BlockSpec((NB, D), lambda i: (i, 0)),
            pl.BlockSpec((NB, NW), lambda i: (i, 0)),
            pl.BlockSpec((NB, 1), lambda i: (i, 0)),
            pl.BlockSpec((H, H), lambda i: (0, 0)),
            pl.BlockSpec((H, H), lambda i: (0, 0)),
            pl.BlockSpec((1, H), lambda i: (0, 0)),
            pl.BlockSpec((H, H), lambda i: (0, 0)),
            pl.BlockSpec((1, H), lambda i: (0, 0)),
            pl.BlockSpec((C, H), lambda i: (0, 0)),
            pl.BlockSpec((1, C), lambda i: (0, 0)),
        ],
        out_specs=pl.BlockSpec((G, C), lambda i: (0, 0)),
        out_shape=jax.ShapeDtypeStruct((G, C), jnp.float32),
        scratch_shapes=[
            pltpu.VMEM((NC, G, D), jnp.float32),
            pltpu.VMEM((NC, G, D), jnp.float32),
            pltpu.VMEM((1, G), jnp.float32),
        ],
    )(sum2, h0, h1, cnt, batch.reshape(N, 1),
      Wl2, Wr2, b2.reshape(1, H), Wm1, bm1.reshape(1, H),
      Wm2, bm2.reshape(1, C))

    return out
